# baseline (speedup 1.0000x reference)
import functools

import jax
import jax.numpy as jnp
from jax import lax
from jax.experimental import pallas as pl
from jax.experimental.pallas import tpu as pltpu
from jax.experimental.pallas import tpu_sc as plsc

_B = 4096
_L = 50
_V = 100000
_D = 128
_R = _B * _L

_NW = 32
_CHUNK = 128
_P = 2
_RP = _R // _P
_ROWS_W = _RP // _NW
_NCHUNK = _ROWS_W // _CHUNK
_NBUF = 5


@functools.partial(
    pl.kernel,
    out_type=jax.ShapeDtypeStruct((_RP, _D), jnp.float32),
    mesh=plsc.VectorSubcoreMesh(core_axis_name="c", subcore_axis_name="s"),
    scratch_types=[
        pltpu.VMEM((_NCHUNK, _CHUNK), jnp.int32),
        pltpu.VMEM((_NBUF, _CHUNK, _D), jnp.float32),
        pltpu.SemaphoreType.DMA((_NBUF,)),
        pltpu.SemaphoreType.DMA((_NBUF,)),
    ],
)
def _sc_gather(tok_hbm, table_hbm, out_hbm, idx_v, rows_v, gsem, ssem):
    w = lax.axis_index("s") * 2 + lax.axis_index("c")
    pltpu.sync_copy(tok_hbm.at[w], idx_v)

    def start_gather(j, s):
        pltpu.async_copy(table_hbm.at[idx_v.at[j]], rows_v.at[s], gsem.at[s])

    for s in range(_NBUF - 1):
        start_gather(s, s)

    def ring_body(jj, _):
        for s in range(_NBUF):
            j = jj * _NBUF + s
            sn = (s + _NBUF - 1) % _NBUF

            @pl.when(j >= 1)
            def _wait_prev_store():
                pltpu.make_async_copy(
                    rows_v.at[sn], out_hbm.at[pl.ds(0, _CHUNK)],
                    ssem.at[sn]).wait()

            @pl.when(j + _NBUF - 1 < _NCHUNK)
            def _next_gather():
                start_gather(j + _NBUF - 1, sn)

            pltpu.make_async_copy(
                table_hbm.at[idx_v.at[j]], rows_v.at[s], gsem.at[s]).wait()
            pltpu.async_copy(
                rows_v.at[s],
                out_hbm.at[pl.ds(w * _ROWS_W + j * _CHUNK, _CHUNK)],
                ssem.at[s])
        return _

    lax.fori_loop(0, _NCHUNK // _NBUF, ring_body, None)
    pltpu.make_async_copy(
        rows_v.at[_NBUF - 1], out_hbm.at[pl.ds(0, _CHUNK)],
        ssem.at[_NBUF - 1]).wait()


def _tc_project_emotions(x, w, bias):
    m = x.shape[0]

    def body(x_ref, w_ref, b_ref, o_ref):
        o_ref[...] = lax.dot_general(
            x_ref[...], w_ref[...], (((1,), (1,)), ((), ())),
            preferred_element_type=jnp.float32) + b_ref[...]

    return pl.pallas_call(
        body,
        grid=(1,),
        in_specs=[
            pl.BlockSpec((m, _D), lambda i: (0, 0)),
            pl.BlockSpec((_D, _D), lambda i: (0, 0)),
            pl.BlockSpec((1, _D), lambda i: (0, 0)),
        ],
        out_specs=pl.BlockSpec((m, _D), lambda i: (0, 0)),
        out_shape=jax.ShapeDtypeStruct((m, _D), jnp.float32),
    )(x, w, bias.reshape(1, _D))


def _tc_project_add(g, emo_proj, we):
    def body(x_ref, emo_ref, w_ref, o_ref):
        o_ref[...] = lax.dot_general(
            x_ref[...], w_ref[...], (((1,), (1,)), ((), ())),
            preferred_element_type=jnp.float32) + emo_ref[...]

    return pl.pallas_call(
        body,
        grid=(_L // _P,),
        in_specs=[
            pl.BlockSpec((_B, _D), lambda i: (i, 0)),
            pl.BlockSpec((_B, _D), lambda i: (0, 0)),
            pl.BlockSpec((_D, _D), lambda i: (0, 0)),
        ],
        out_specs=pl.BlockSpec((_B, _D), lambda i: (i, 0)),
        out_shape=jax.ShapeDtypeStruct((_RP, _D), jnp.float32),
    )(g, emo_proj, we)


def kernel(tokens, emotions, table, W, b):
    tokens = tokens.astype(jnp.int32)
    we = W[:, :_D]
    wm = W[:, _D:]

    emo_proj = _tc_project_emotions(emotions, wm, b)
    tok_t = tokens.T.reshape(_P, _NW, _NCHUNK, _CHUNK)
    parts = []
    for p in range(_P):
        g = _sc_gather(tok_t[p], table)
        parts.append(_tc_project_add(g, emo_proj, we))
    out = jnp.concatenate(parts, axis=0)
    return out.reshape(_L, _B, _D).transpose(1, 0, 2)


# speedup vs baseline: 1.3855x; 1.3855x over previous
import functools

import jax
import jax.numpy as jnp
from jax import lax
from jax.experimental import pallas as pl
from jax.experimental.pallas import tpu as pltpu
from jax.experimental.pallas import tpu_sc as plsc

_B = 4096
_L = 50
_V = 100000
_D = 128
_R = _B * _L

_NW = 32
_CHUNK = 128
_P = 2
_RP = _R // _P
_ROWS_W = _RP // _NW
_NCHUNK = _ROWS_W // _CHUNK
_NBUF = 5


@functools.partial(
    pl.kernel,
    out_type=jax.ShapeDtypeStruct((_RP, _D), jnp.float32),
    mesh=plsc.VectorSubcoreMesh(core_axis_name="c", subcore_axis_name="s"),
    scratch_types=[
        pltpu.VMEM((_NCHUNK, _CHUNK), jnp.int32),
        pltpu.VMEM((_NBUF, _CHUNK, _D), jnp.float32),
        pltpu.SemaphoreType.DMA((_NBUF,)),
        pltpu.SemaphoreType.DMA((_NBUF,)),
    ],
)
def _sc_gather(tok_hbm, table_hbm, out_hbm, idx_v, rows_v, gsem, ssem):
    w = lax.axis_index("s") * 2 + lax.axis_index("c")
    pltpu.sync_copy(tok_hbm.at[w], idx_v)

    def start_gather(j, s):
        pltpu.async_copy(table_hbm.at[idx_v.at[j]], rows_v.at[s], gsem.at[s])

    for s in range(_NBUF - 1):
        start_gather(s, s)

    def ring_body(jj, _):
        for s in range(_NBUF):
            j = jj * _NBUF + s
            sn = (s + _NBUF - 1) % _NBUF

            @pl.when(j >= 1)
            def _wait_prev_store():
                pltpu.make_async_copy(
                    rows_v.at[sn], out_hbm.at[pl.ds(0, _CHUNK)],
                    ssem.at[sn]).wait()

            @pl.when(j + _NBUF - 1 < _NCHUNK)
            def _next_gather():
                start_gather(j + _NBUF - 1, sn)

            pltpu.make_async_copy(
                table_hbm.at[idx_v.at[j]], rows_v.at[s], gsem.at[s]).wait()
            pltpu.async_copy(
                rows_v.at[s],
                out_hbm.at[pl.ds(w * _ROWS_W + j * _CHUNK, _CHUNK)],
                ssem.at[s])
        return _

    lax.fori_loop(0, _NCHUNK // _NBUF, ring_body, None)
    pltpu.make_async_copy(
        rows_v.at[_NBUF - 1], out_hbm.at[pl.ds(0, _CHUNK)],
        ssem.at[_NBUF - 1]).wait()


def _tc_project_emotions(x, w, bias):
    m = x.shape[0]

    def body(x_ref, w_ref, b_ref, o_ref):
        o_ref[...] = lax.dot_general(
            x_ref[...], w_ref[...], (((1,), (1,)), ((), ())),
            preferred_element_type=jnp.float32) + b_ref[...]

    return pl.pallas_call(
        body,
        grid=(1,),
        in_specs=[
            pl.BlockSpec((m, _D), lambda i: (0, 0)),
            pl.BlockSpec((_D, _D), lambda i: (0, 0)),
            pl.BlockSpec((1, _D), lambda i: (0, 0)),
        ],
        out_specs=pl.BlockSpec((m, _D), lambda i: (0, 0)),
        out_shape=jax.ShapeDtypeStruct((m, _D), jnp.float32),
    )(x, w, bias.reshape(1, _D))


def _tc_project_add_first(g, emo_proj, we):
    def body(x_ref, emo_ref, w_ref, o_ref):
        o_ref[...] = lax.dot_general(
            x_ref[...], w_ref[...], (((1,), (1,)), ((), ())),
            preferred_element_type=jnp.float32) + emo_ref[...]

    return pl.pallas_call(
        body,
        grid=(_L // _P,),
        in_specs=[
            pl.BlockSpec((_B, _D), lambda i: (i, 0)),
            pl.BlockSpec((_B, _D), lambda i: (0, 0)),
            pl.BlockSpec((_D, _D), lambda i: (0, 0)),
        ],
        out_specs=pl.BlockSpec((_B, _D), lambda i: (i, 0)),
        out_shape=jax.ShapeDtypeStruct((_R, _D), jnp.float32),
    )(g, emo_proj, we)


def _tc_project_add_part(g, emo_proj, we, dst, part):
    base = part * (_L // _P)

    def body(x_ref, emo_ref, w_ref, dst_ref, o_ref):
        del dst_ref
        o_ref[...] = lax.dot_general(
            x_ref[...], w_ref[...], (((1,), (1,)), ((), ())),
            preferred_element_type=jnp.float32) + emo_ref[...]

    return pl.pallas_call(
        body,
        grid=(_L // _P,),
        in_specs=[
            pl.BlockSpec((_B, _D), lambda i: (i, 0)),
            pl.BlockSpec((_B, _D), lambda i: (0, 0)),
            pl.BlockSpec((_D, _D), lambda i: (0, 0)),
            pl.BlockSpec((8, _D), lambda i: (0, 0)),
        ],
        out_specs=pl.BlockSpec((_B, _D), lambda i, base=base: (base + i, 0)),
        out_shape=jax.ShapeDtypeStruct((_R, _D), jnp.float32),
        input_output_aliases={3: 0},
    )(g, emo_proj, we, dst)


def kernel(tokens, emotions, table, W, b):
    tokens = tokens.astype(jnp.int32)
    we = W[:, :_D]
    wm = W[:, _D:]

    emo_proj = _tc_project_emotions(emotions, wm, b)
    tok_t = tokens.T.reshape(_P, _NW, _NCHUNK, _CHUNK)
    gs = [_sc_gather(tok_t[p], table) for p in range(_P)]
    out = _tc_project_add_first(gs[0], emo_proj, we)
    for p in range(1, _P):
        out = _tc_project_add_part(gs[p], emo_proj, we, out, p)
    return out.reshape(_L, _B, _D).transpose(1, 0, 2)


# speedup vs baseline: 1.4597x; 1.0535x over previous
import functools

import jax
import jax.numpy as jnp
from jax import lax
from jax.experimental import pallas as pl
from jax.experimental.pallas import tpu as pltpu
from jax.experimental.pallas import tpu_sc as plsc

_B = 4096
_L = 50
_V = 100000
_D = 128
_R = _B * _L

_NW = 32
_BW = _B // _NW
_NCHUNK = _L
_NBUF = 5


@functools.partial(
    pl.kernel,
    out_type=jax.ShapeDtypeStruct((_R, _D), jnp.float32),
    mesh=plsc.VectorSubcoreMesh(core_axis_name="c", subcore_axis_name="s"),
    scratch_types=[
        pltpu.VMEM((_NCHUNK, _BW), jnp.int32),
        pltpu.VMEM((_BW, _D), jnp.float32),
        pltpu.VMEM((_NBUF, _BW, _D), jnp.float32),
        pltpu.SemaphoreType.DMA((_NBUF,)),
        pltpu.SemaphoreType.DMA((_NBUF,)),
    ],
)
def _sc_gather_add(tok_hbm, emo_hbm, proj_hbm, out_hbm,
                   idx_v, emo_v, rows_v, gsem, ssem):
    w = lax.axis_index("s") * 2 + lax.axis_index("c")
    pltpu.sync_copy(tok_hbm.at[w], idx_v)
    pltpu.sync_copy(emo_hbm.at[pl.ds(w * _BW, _BW)], emo_v)

    def start_gather(j, s):
        pltpu.async_copy(proj_hbm.at[idx_v.at[j]], rows_v.at[s], gsem.at[s])

    for s in range(_NBUF - 1):
        start_gather(s, s)

    def ring_body(jj, _):
        for s in range(_NBUF):
            j = jj * _NBUF + s
            sn = (s + _NBUF - 1) % _NBUF

            @pl.when(j >= 1)
            def _wait_prev_store():
                pltpu.make_async_copy(
                    rows_v.at[sn], out_hbm.at[pl.ds(0, _BW)],
                    ssem.at[sn]).wait()

            @pl.when(j + _NBUF - 1 < _NCHUNK)
            def _next_gather():
                start_gather(j + _NBUF - 1, sn)

            pltpu.make_async_copy(
                proj_hbm.at[idx_v.at[j]], rows_v.at[s], gsem.at[s]).wait()

            def row_body(r, _, s=s):
                for k in range(_D // 16):
                    e = emo_v[r, pl.ds(k * 16, 16)]
                    plsc.addupdate(rows_v.at[s, r, pl.ds(k * 16, 16)], e)
                return _

            lax.fori_loop(0, _BW, row_body, None)
            pltpu.async_copy(
                rows_v.at[s],
                out_hbm.at[pl.ds(j * _B + w * _BW, _BW)],
                ssem.at[s])
        return _

    lax.fori_loop(0, _NCHUNK // _NBUF, ring_body, None)
    pltpu.make_async_copy(
        rows_v.at[_NBUF - 1], out_hbm.at[pl.ds(0, _BW)],
        ssem.at[_NBUF - 1]).wait()


def _tc_project_table(x, w):
    m = x.shape[0]
    blk = 4000

    def body(x_ref, w_ref, o_ref):
        o_ref[...] = lax.dot_general(
            x_ref[...], w_ref[...], (((1,), (1,)), ((), ())),
            preferred_element_type=jnp.float32)

    return pl.pallas_call(
        body,
        grid=(m // blk,),
        in_specs=[
            pl.BlockSpec((blk, _D), lambda i: (i, 0)),
            pl.BlockSpec((_D, _D), lambda i: (0, 0)),
        ],
        out_specs=pl.BlockSpec((blk, _D), lambda i: (i, 0)),
        out_shape=jax.ShapeDtypeStruct((m, _D), jnp.float32),
    )(x, w)


def _tc_project_emotions(x, w, bias):
    m = x.shape[0]

    def body(x_ref, w_ref, b_ref, o_ref):
        o_ref[...] = lax.dot_general(
            x_ref[...], w_ref[...], (((1,), (1,)), ((), ())),
            preferred_element_type=jnp.float32) + b_ref[...]

    return pl.pallas_call(
        body,
        grid=(1,),
        in_specs=[
            pl.BlockSpec((m, _D), lambda i: (0, 0)),
            pl.BlockSpec((_D, _D), lambda i: (0, 0)),
            pl.BlockSpec((1, _D), lambda i: (0, 0)),
        ],
        out_specs=pl.BlockSpec((m, _D), lambda i: (0, 0)),
        out_shape=jax.ShapeDtypeStruct((m, _D), jnp.float32),
    )(x, w, bias.reshape(1, _D))


def kernel(tokens, emotions, table, W, b):
    tokens = tokens.astype(jnp.int32)
    we = W[:, :_D]
    wm = W[:, _D:]

    proj = _tc_project_table(table, we)
    emo_proj = _tc_project_emotions(emotions, wm, b)
    tok_w = tokens.T.reshape(_L, _NW, _BW).transpose(1, 0, 2)
    out = _sc_gather_add(tok_w, emo_proj, proj)
    return out.reshape(_L, _B, _D).transpose(1, 0, 2)
